# Initial kernel scaffold; baseline (speedup 1.0000x reference)
import jax
import jax.numpy as jnp
from jax.experimental import pallas as pl


def kernel(x, s_log, alpha_log, delta_log, r_log):
    raise NotImplementedError("write your pallas kernel here")


# speedup vs baseline: 6.8759x; 6.8759x over previous
import jax
import jax.numpy as jnp
from jax.experimental import pallas as pl
from jax.experimental.pallas import tpu as pltpu

_EPS = 1e-05
_LANE = 128


def _pcen_body(x_ref, s_ref, al_ref, dl_ref, rl_ref, o_ref):
    F = x_ref.shape[1]
    T = x_ref.shape[2]
    n_full = T // _LANE
    rem = T - n_full * _LANE

    s_col = jnp.exp(s_ref[0])
    log_a = jnp.log1p(-s_col)
    alpha = jnp.exp(al_ref[...])
    r_col = jnp.exp(rl_ref[...])
    delta = jnp.exp(dl_ref[...])
    delta_r = jnp.exp(r_col * dl_ref[...])
    neg_alpha = -alpha

    lane = jax.lax.broadcasted_iota(jnp.int32, (F, _LANE), 1)
    lane_f = lane.astype(jnp.float32)
    pw = jnp.exp(log_a * (lane_f + 1.0))
    shifts = (1, 2, 4, 8, 16, 32, 64)
    m = {d: jnp.where(lane >= d, pw[:, d - 1:d], 0.0) for d in shifts}

    eps = jnp.float32(_EPS)

    def scan_tile(z):
        c = z
        for d in shifts:
            rolled = pltpu.roll(c, d, axis=1)
            c = c + rolled * m[d]
        return c

    def pcen_tile(xt, y):
        sm = jnp.exp(neg_alpha * jnp.log(y + eps))
        return jnp.exp(r_col * jnp.log(xt * sm + delta)) - delta_r

    y_prev = None
    y_carry_tail = None
    for ti in range(n_full):
        lo = ti * _LANE
        xt = x_ref[0, :, lo:lo + _LANE]
        z = xt * s_col
        if ti == 0:
            z = jnp.where(lane == 0, xt, z)
        c = scan_tile(z)
        y = c if y_prev is None else c + pw * y_prev
        y_prev = y[:, _LANE - 1:_LANE]
        if rem and ti == n_full - 1:
            y_carry_tail = y[:, rem - 1:rem]
        o_ref[0, 0, :, lo:lo + _LANE] = pcen_tile(xt, y)

    if rem:
        lo = T - _LANE
        xt = x_ref[0, :, lo:lo + _LANE]
        z = xt * s_col
        c = scan_tile(z)
        y = c + pw * y_carry_tail
        p = pcen_tile(xt, y)
        o_ref[0, 0, :, n_full * _LANE:T] = p[:, _LANE - rem:_LANE]


def kernel(x, s_log, alpha_log, delta_log, r_log):
    B, F, T = x.shape
    K = s_log.shape[0]
    s3 = s_log.reshape(K, F, 1)
    al = alpha_log.reshape(F, 1)
    dl = delta_log.reshape(F, 1)
    rl = r_log.reshape(F, 1)

    return pl.pallas_call(
        _pcen_body,
        out_shape=jax.ShapeDtypeStruct((B, K, F, T), x.dtype),
        grid=(B, K),
        in_specs=[
            pl.BlockSpec((1, F, T), lambda b, k: (b, 0, 0)),
            pl.BlockSpec((1, F, 1), lambda b, k: (k, 0, 0)),
            pl.BlockSpec((F, 1), lambda b, k: (0, 0)),
            pl.BlockSpec((F, 1), lambda b, k: (0, 0)),
            pl.BlockSpec((F, 1), lambda b, k: (0, 0)),
        ],
        out_specs=pl.BlockSpec((1, 1, F, T), lambda b, k: (b, k, 0, 0)),
        compiler_params=pltpu.CompilerParams(
            dimension_semantics=("parallel", "arbitrary"),
        ),
        name="tmrpcen",
    )(x, s3, al, dl, rl)


# speedup vs baseline: 13.1483x; 1.9122x over previous
import jax
import jax.numpy as jnp
from jax.experimental import pallas as pl
from jax.experimental.pallas import tpu as pltpu

_EPS = 1e-05
_LANE = 128


def _pcen_body(x_ref, s_ref, al_ref, dl_ref, rl_ref, o_ref):
    F = x_ref.shape[1]
    T = x_ref.shape[2]
    n_full = T // _LANE
    rem = T - n_full * _LANE

    s_col = jnp.exp(s_ref[0])
    log_a = jnp.log1p(-s_col)
    alpha = jnp.exp(al_ref[...])
    r_col = jnp.exp(rl_ref[...])
    delta = jnp.exp(dl_ref[...])
    delta_r = jnp.exp(r_col * dl_ref[...])
    neg_alpha = -alpha

    lane = jax.lax.broadcasted_iota(jnp.int32, (F, _LANE), 1)
    lane_f = lane.astype(jnp.float32)
    pw = jnp.exp(log_a * (lane_f + 1.0))
    shifts = (1, 2, 4, 8, 16, 32, 64)
    m = {d: jnp.where(lane >= d, pw[:, d - 1:d], 0.0) for d in shifts}

    eps = jnp.float32(_EPS)

    def scan_group(cs):
        for d in shifts:
            md = m[d]
            cs = [c + pltpu.roll(c, d, axis=1) * md for c in cs]
        return cs

    def pcen_tile(xt, y):
        sm = jnp.exp2(neg_alpha * jnp.log2(y + eps))
        return jnp.exp2(r_col * jnp.log2(xt * sm + delta)) - delta_r

    _GROUP = 8
    y_prev = None
    y_carry_tail = None
    for g0 in range(0, n_full, _GROUP):
        g1 = min(g0 + _GROUP, n_full)
        xts = [x_ref[0, :, ti * _LANE:(ti + 1) * _LANE] for ti in range(g0, g1)]
        cs = [xt * s_col for xt in xts]
        if g0 == 0:
            cs[0] = jnp.where(lane == 0, xts[0], cs[0])
        cs = scan_group(cs)
        for i, ti in enumerate(range(g0, g1)):
            y = cs[i] if y_prev is None else cs[i] + pw * y_prev
            y_prev = y[:, _LANE - 1:_LANE]
            if rem and ti == n_full - 1:
                y_carry_tail = y[:, rem - 1:rem]
            lo = ti * _LANE
            o_ref[0, 0, :, lo:lo + _LANE] = pcen_tile(xts[i], y)

    if rem:
        lo = T - _LANE
        xt = x_ref[0, :, lo:lo + _LANE]
        (c,) = scan_group([xt * s_col])
        y = c + pw * y_carry_tail
        p = pcen_tile(xt, y)
        o_ref[0, 0, :, n_full * _LANE:T] = p[:, _LANE - rem:_LANE]


def kernel(x, s_log, alpha_log, delta_log, r_log):
    B, F, T = x.shape
    K = s_log.shape[0]
    s3 = s_log.reshape(K, F, 1)
    al = alpha_log.reshape(F, 1)
    dl = delta_log.reshape(F, 1)
    rl = r_log.reshape(F, 1)

    return pl.pallas_call(
        _pcen_body,
        out_shape=jax.ShapeDtypeStruct((B, K, F, T), x.dtype),
        grid=(B, K),
        in_specs=[
            pl.BlockSpec((1, F, T), lambda b, k: (b, 0, 0)),
            pl.BlockSpec((1, F, 1), lambda b, k: (k, 0, 0)),
            pl.BlockSpec((F, 1), lambda b, k: (0, 0)),
            pl.BlockSpec((F, 1), lambda b, k: (0, 0)),
            pl.BlockSpec((F, 1), lambda b, k: (0, 0)),
        ],
        out_specs=pl.BlockSpec((1, 1, F, T), lambda b, k: (b, k, 0, 0)),
        compiler_params=pltpu.CompilerParams(
            dimension_semantics=("parallel", "arbitrary"),
        ),
        name="tmrpcen",
    )(x, s3, al, dl, rl)


# speedup vs baseline: 24.0771x; 1.8312x over previous
import jax
import jax.numpy as jnp
from jax.experimental import pallas as pl
from jax.experimental.pallas import tpu as pltpu

_EPS = 1e-05
_LANE = 128


def _pcen_body(x_ref, s_ref, al_ref, dl_ref, rl_ref, o_ref):
    F = x_ref.shape[1]
    T = x_ref.shape[2]
    n_full = T // _LANE
    rem = T - n_full * _LANE

    s_col = jnp.exp(s_ref[0])
    log_a = jnp.log1p(-s_col)
    alpha = jnp.exp(al_ref[...])
    r_col = jnp.exp(rl_ref[...])
    delta = jnp.exp(dl_ref[...])
    delta_r = jnp.exp(r_col * dl_ref[...])
    neg_alpha = -alpha

    lane = jax.lax.broadcasted_iota(jnp.int32, (F, _LANE), 1)
    lane_f = lane.astype(jnp.float32)
    inv_ln2 = jnp.float32(1.4426950408889634)
    log2_a = log_a * inv_ln2
    lmod = lane_f - jnp.floor(lane_f * (1.0 / 64.0)) * 64.0
    pw = jnp.exp2(log2_a * (lane_f + 1.0))
    pw0 = jnp.exp2(log2_a * lmod)
    ipw = jnp.exp2(-log2_a * lmod)
    sipw = s_col * ipw
    phw = jnp.where(lane >= 64, jnp.exp2(log2_a * (lane_f - 63.0)), 0.0)
    jrow = jax.lax.broadcasted_iota(jnp.int32, (_LANE, _LANE), 0)
    tcol = jax.lax.broadcasted_iota(jnp.int32, (_LANE, _LANE), 1)
    scan_m = ((jrow <= tcol) & ((jrow // 64) == (tcol // 64))
              ).astype(jnp.bfloat16)

    eps = jnp.float32(_EPS)

    def scan_tile(xt, first, carry):
        u = xt * sipw
        if first:
            u = jnp.where(lane == 0, xt, u)
        uh = u.astype(jnp.bfloat16)
        ul = (u - uh.astype(jnp.float32)).astype(jnp.bfloat16)
        g = (jnp.dot(uh, scan_m, preferred_element_type=jnp.float32)
             + jnp.dot(ul, scan_m, preferred_element_type=jnp.float32))
        c = g * pw0
        e0 = c[:, 63:64]
        y = c + phw * e0
        if carry is not None:
            y = y + pw * carry
        return y

    def pcen_tile(xt, y):
        sm = jnp.exp2(neg_alpha * jnp.log2(y + eps))
        return jnp.exp2(r_col * jnp.log2(xt * sm + delta)) - delta_r

    y_prev = None
    y_carry_tail = None
    for ti in range(n_full):
        lo = ti * _LANE
        xt = x_ref[0, :, lo:lo + _LANE]
        y = scan_tile(xt, ti == 0, y_prev)
        y_prev = y[:, _LANE - 1:_LANE]
        if rem and ti == n_full - 1:
            y_carry_tail = y[:, rem - 1:rem]
        o_ref[0, 0, :, lo:lo + _LANE] = pcen_tile(xt, y)

    if rem:
        lo = T - _LANE
        xt = x_ref[0, :, lo:lo + _LANE]
        y = scan_tile(xt, False, y_carry_tail)
        p = pcen_tile(xt, y)
        o_ref[0, 0, :, n_full * _LANE:T] = p[:, _LANE - rem:_LANE]


def kernel(x, s_log, alpha_log, delta_log, r_log):
    B, F, T = x.shape
    K = s_log.shape[0]
    s3 = s_log.reshape(K, F, 1)
    al = alpha_log.reshape(F, 1)
    dl = delta_log.reshape(F, 1)
    rl = r_log.reshape(F, 1)

    return pl.pallas_call(
        _pcen_body,
        out_shape=jax.ShapeDtypeStruct((B, K, F, T), x.dtype),
        grid=(B, K),
        in_specs=[
            pl.BlockSpec((1, F, T), lambda b, k: (b, 0, 0)),
            pl.BlockSpec((1, F, 1), lambda b, k: (k, 0, 0)),
            pl.BlockSpec((F, 1), lambda b, k: (0, 0)),
            pl.BlockSpec((F, 1), lambda b, k: (0, 0)),
            pl.BlockSpec((F, 1), lambda b, k: (0, 0)),
        ],
        out_specs=pl.BlockSpec((1, 1, F, T), lambda b, k: (b, k, 0, 0)),
        compiler_params=pltpu.CompilerParams(
            dimension_semantics=("parallel", "arbitrary"),
        ),
        name="tmrpcen",
    )(x, s3, al, dl, rl)


# speedup vs baseline: 24.3205x; 1.0101x over previous
import numpy as np
import jax
import jax.numpy as jnp
from jax.experimental import pallas as pl
from jax.experimental.pallas import tpu as pltpu

_EPS = 1e-05
_LANE = 128
_BLK = 64


def _pcen_body(x_ref, sipw_ref, pw0_ref, pw_ref, phw_ref, m_ref,
               nal_ref, r_ref, d_ref, dr_ref, o_ref):
    F = x_ref.shape[1]
    T = x_ref.shape[2]
    n_full = T // _LANE
    rem = T - n_full * _LANE

    sipw = sipw_ref[0]
    pw0 = pw0_ref[0]
    pw = pw_ref[0]
    phw = phw_ref[0]
    scan_m = m_ref[...]
    neg_alpha = nal_ref[...]
    r_col = r_ref[...]
    delta = d_ref[...]
    delta_r = dr_ref[...]

    lane = jax.lax.broadcasted_iota(jnp.int32, (F, _LANE), 1)
    eps = jnp.float32(_EPS)

    def scan_tile(xt, first, carry):
        u = xt * sipw
        if first:
            u = jnp.where(lane == 0, xt, u)
        uh = u.astype(jnp.bfloat16)
        ul = (u - uh.astype(jnp.float32)).astype(jnp.bfloat16)
        g = (jnp.dot(uh, scan_m, preferred_element_type=jnp.float32)
             + jnp.dot(ul, scan_m, preferred_element_type=jnp.float32))
        c = g * pw0
        e0 = c[:, _BLK - 1:_BLK]
        y = c + phw * e0
        if carry is not None:
            y = y + pw * carry
        return y

    def pcen_tile(xt, y):
        sm = jnp.exp2(neg_alpha * jnp.log2(y + eps))
        return jnp.exp2(r_col * jnp.log2(xt * sm + delta)) - delta_r

    y_prev = None
    y_carry_tail = None
    for ti in range(n_full):
        lo = ti * _LANE
        xt = x_ref[0, :, lo:lo + _LANE]
        y = scan_tile(xt, ti == 0, y_prev)
        y_prev = y[:, _LANE - 1:_LANE]
        if rem and ti == n_full - 1:
            y_carry_tail = y[:, rem - 1:rem]
        o_ref[0, 0, :, lo:lo + _LANE] = pcen_tile(xt, y)

    if rem:
        lo = T - _LANE
        xt = x_ref[0, :, lo:lo + _LANE]
        y = scan_tile(xt, False, y_carry_tail)
        p = pcen_tile(xt, y)
        o_ref[0, 0, :, n_full * _LANE:T] = p[:, _LANE - rem:_LANE]


def kernel(x, s_log, alpha_log, delta_log, r_log):
    B, F, T = x.shape
    K = s_log.shape[0]

    s = jnp.exp(s_log)
    log2_a = jnp.log1p(-s) * jnp.float32(1.4426950408889634)
    a2 = log2_a[:, :, None]
    l = jnp.arange(_LANE, dtype=jnp.float32)
    lmod = l - jnp.floor(l * (1.0 / _BLK)) * _BLK
    pw = jnp.exp2(a2 * (l + 1.0))
    pw0 = jnp.exp2(a2 * lmod)
    sipw = s[:, :, None] * jnp.exp2(-a2 * lmod)
    phw = jnp.where(l >= _BLK, jnp.exp2(a2 * (l - (_BLK - 1.0))), 0.0)

    r = jnp.exp(r_log)
    nal = (-jnp.exp(alpha_log)).reshape(F, 1)
    rr = r.reshape(F, 1)
    dd = jnp.exp(delta_log).reshape(F, 1)
    dr = jnp.exp(r * delta_log).reshape(F, 1)

    jrow, tcol = np.indices((_LANE, _LANE))
    scan_m = jnp.asarray(
        (jrow <= tcol) & ((jrow // _BLK) == (tcol // _BLK)),
        dtype=jnp.bfloat16)

    ktab = pl.BlockSpec((1, F, _LANE), lambda b, k: (k, 0, 0))
    fcol = pl.BlockSpec((F, 1), lambda b, k: (0, 0))
    return pl.pallas_call(
        _pcen_body,
        out_shape=jax.ShapeDtypeStruct((B, K, F, T), x.dtype),
        grid=(B, K),
        in_specs=[
            pl.BlockSpec((1, F, T), lambda b, k: (b, 0, 0)),
            ktab, ktab, ktab, ktab,
            pl.BlockSpec((_LANE, _LANE), lambda b, k: (0, 0)),
            fcol, fcol, fcol, fcol,
        ],
        out_specs=pl.BlockSpec((1, 1, F, T), lambda b, k: (b, k, 0, 0)),
        compiler_params=pltpu.CompilerParams(
            dimension_semantics=("parallel", "arbitrary"),
        ),
        name="tmrpcen",
    )(x, sipw, pw0, pw, phw, scan_m, nal, rr, dd, dr)
